# Initial kernel scaffold; baseline (speedup 1.0000x reference)
import jax
import jax.numpy as jnp
from jax.experimental import pallas as pl


def kernel(batch_targets, bucket_boundaries, class_weights):
    raise NotImplementedError("write your pallas kernel here")


# speedup vs baseline: 3.2974x; 3.2974x over previous
import functools

import jax
import jax.numpy as jnp
from jax import lax
from jax.experimental import pallas as pl
from jax.experimental.pallas import tpu as pltpu
from jax.experimental.pallas import tpu_sc as plsc

_M = 1024
_LANES = 16


@functools.cache
def _build_sc_kernel(n_total: int, chunk: int):
    info = plsc.get_sparse_core_info()
    nc, ns = info.num_cores, info.num_subcores
    nw = nc * ns
    ew = n_total // nw
    nchunks = ew // chunk
    nvec = chunk // _LANES
    mesh = plsc.VectorSubcoreMesh(core_axis_name="c", subcore_axis_name="s")

    @functools.partial(
        pl.kernel,
        mesh=mesh,
        compiler_params=pltpu.CompilerParams(
            use_tc_tiling_on_sc=False, needs_layout_passes=False),
        out_type=jax.ShapeDtypeStruct((n_total,), jnp.float32),
        scratch_types=[
            pltpu.VMEM((_M,), jnp.float32),
            pltpu.VMEM((2 * _M,), jnp.float32),
            pltpu.VMEM((chunk,), jnp.float32),
            pltpu.VMEM((chunk,), jnp.float32),
        ],
    )
    def sc_kernel(t_hbm, thr_hbm, w2_hbm, out_hbm, thr_v, w2_v, in_v, out_v):
        wid = lax.axis_index("s") * nc + lax.axis_index("c")
        base = wid * ew
        pltpu.sync_copy(thr_hbm, thr_v)
        pltpu.sync_copy(w2_hbm, w2_v)

        def chunk_body(g, carry):
            off = base + g * chunk
            pltpu.sync_copy(t_hbm.at[pl.ds(off, chunk)], in_v)

            def vec_body(i, c2):
                t = in_v[pl.ds(i * _LANES, _LANES)]
                m = jnp.clip((t * float(_M)).astype(jnp.int32), 0, _M - 1)
                th = plsc.load_gather(thr_v, [m])
                j = 2 * m + jnp.where(t > th, jnp.int32(1), jnp.int32(0))
                out_v[pl.ds(i * _LANES, _LANES)] = plsc.load_gather(w2_v, [j])
                return c2

            lax.fori_loop(0, nvec, vec_body, 0)
            pltpu.sync_copy(out_v, out_hbm.at[pl.ds(off, chunk)])
            return carry

        lax.fori_loop(0, nchunks, chunk_body, 0)

    return sc_kernel


def _build_luts(bucket_boundaries, class_weights):
    b = bucket_boundaries.shape[0]
    k = class_weights.shape[0]
    grid = jnp.arange(_M, dtype=jnp.float32) * jnp.float32(1.0 / _M)
    c0 = jnp.searchsorted(bucket_boundaries, grid, side="left").astype(jnp.int32)
    thr = jnp.where(c0 < b,
                    bucket_boundaries[jnp.clip(c0, 0, b - 1)],
                    jnp.float32(2.0))
    fw = lambda c: class_weights[jnp.clip(c - 1, 0, k - 1)]
    w_lo = fw(c0)
    w_hi = jnp.where(c0 < b, fw(c0 + 1), w_lo)
    w2 = jnp.stack([w_lo, w_hi], axis=1).reshape(-1)
    return thr, w2


def kernel(batch_targets, bucket_boundaries, class_weights):
    n = batch_targets.shape[0]
    thr, w2 = _build_luts(bucket_boundaries, class_weights)
    out = _build_sc_kernel(n, 16384)(batch_targets, thr, w2)
    return out[:, None]


# speedup vs baseline: 5.5611x; 1.6865x over previous
import functools

import jax
import jax.numpy as jnp
from jax import lax
from jax.experimental import pallas as pl
from jax.experimental.pallas import tpu as pltpu
from jax.experimental.pallas import tpu_sc as plsc

_M = 1024
_LANES = 16


@functools.cache
def _build_sc_kernel(n_total: int, chunk: int):
    info = plsc.get_sparse_core_info()
    nc, ns = info.num_cores, info.num_subcores
    nw = nc * ns
    ew = n_total // nw
    nchunks = ew // chunk
    nvec = chunk // _LANES
    mesh = plsc.VectorSubcoreMesh(core_axis_name="c", subcore_axis_name="s")

    @functools.partial(
        pl.kernel,
        mesh=mesh,
        compiler_params=pltpu.CompilerParams(
            use_tc_tiling_on_sc=False, needs_layout_passes=False),
        out_type=jax.ShapeDtypeStruct((n_total,), jnp.float32),
        scratch_types=[
            pltpu.VMEM((_M,), jnp.float32),
            pltpu.VMEM((2 * _M,), jnp.float32),
            pltpu.VMEM((chunk,), jnp.float32),
            pltpu.VMEM((chunk,), jnp.float32),
        ],
    )
    def sc_kernel(t_hbm, thr_hbm, w2_hbm, out_hbm, thr_v, w2_v, in_v, out_v):
        wid = lax.axis_index("s") * nc + lax.axis_index("c")
        base = wid * ew
        pltpu.sync_copy(thr_hbm, thr_v)
        pltpu.sync_copy(w2_hbm, w2_v)

        def chunk_body(g, carry):
            off = base + g * chunk
            pltpu.sync_copy(t_hbm.at[pl.ds(off, chunk)], in_v)

            @plsc.parallel_loop(0, chunk, _LANES, unroll=8)
            def _(i):
                t = in_v[pl.ds(i, _LANES)]
                m = jnp.clip((t * float(_M)).astype(jnp.int32), 0, _M - 1)
                th = plsc.load_gather(thr_v, [m])
                j = 2 * m + jnp.where(t > th, jnp.int32(1), jnp.int32(0))
                out_v[pl.ds(i, _LANES)] = plsc.load_gather(w2_v, [j])

            pltpu.sync_copy(out_v, out_hbm.at[pl.ds(off, chunk)])
            return carry

        lax.fori_loop(0, nchunks, chunk_body, 0)

    return sc_kernel


def _build_luts(bucket_boundaries, class_weights):
    b = bucket_boundaries.shape[0]
    k = class_weights.shape[0]
    grid = jnp.arange(_M, dtype=jnp.float32) * jnp.float32(1.0 / _M)
    c0 = jnp.searchsorted(bucket_boundaries, grid, side="left").astype(jnp.int32)
    thr = jnp.where(c0 < b,
                    bucket_boundaries[jnp.clip(c0, 0, b - 1)],
                    jnp.float32(2.0))
    fw = lambda c: class_weights[jnp.clip(c - 1, 0, k - 1)]
    w_lo = fw(c0)
    w_hi = jnp.where(c0 < b, fw(c0 + 1), w_lo)
    w2 = jnp.stack([w_lo, w_hi], axis=1).reshape(-1)
    return thr, w2


def kernel(batch_targets, bucket_boundaries, class_weights):
    n = batch_targets.shape[0]
    thr, w2 = _build_luts(bucket_boundaries, class_weights)
    out = _build_sc_kernel(n, 16384)(batch_targets, thr, w2)
    return out[:, None]
